# Initial kernel scaffold; baseline (speedup 1.0000x reference)
import jax
import jax.numpy as jnp
from jax.experimental import pallas as pl


def kernel(quant_input, weight):
    raise NotImplementedError("write your pallas kernel here")


# speedup vs baseline: 2.5600x; 2.5600x over previous
import jax
import jax.numpy as jnp
from jax.experimental import pallas as pl

_B, _C, _H, _W, _K = 8, 64, 64, 64, 1024
_HW = _H * _W
_T = 1024
_NB = _HW // _T


def _vq_block(x_ref, w_ref, out_ref, idx_ref, loss_ref):
    b = pl.program_id(0)
    nb = pl.program_id(1)
    x = x_ref[0]
    w = w_ref[...]
    s = jnp.dot(w, x, preferred_element_type=jnp.float32)
    b2 = jnp.sum(w * w, axis=1, keepdims=True)
    m = b2 - 2.0 * s
    minm = jnp.min(m, axis=0, keepdims=True)
    kiota = jax.lax.broadcasted_iota(jnp.int32, m.shape, 0)
    idx = jnp.min(jnp.where(m == minm, kiota, _K), axis=0,
                  keepdims=True)
    idx_ref[0, 0] = idx
    oh = (kiota == idx).astype(jnp.float32)
    g = jax.lax.dot_general(w, oh, (((0,), (0,)), ((), ())),
                            preferred_element_type=jnp.float32)
    out_ref[0] = g
    a2 = jnp.sum(x * x, axis=0, keepdims=True)

    @pl.when(jnp.logical_and(b == 0, nb == 0))
    def _init():
        loss_ref[...] = jnp.zeros_like(loss_ref)

    loss_ref[...] += jnp.sum(minm + a2, axis=(0, 1), keepdims=True)


def kernel(quant_input, weight):
    x = quant_input.reshape(_B, _C, _HW)
    out, idx, loss = pl.pallas_call(
        _vq_block,
        grid=(_B, _NB),
        in_specs=[
            pl.BlockSpec((1, _C, _T), lambda b, n: (b, 0, n)),
            pl.BlockSpec((_K, _C), lambda b, n: (0, 0)),
        ],
        out_specs=[
            pl.BlockSpec((1, _C, _T), lambda b, n: (b, 0, n)),
            pl.BlockSpec((1, 1, 1, _T), lambda b, n: (b, n, 0, 0)),
            pl.BlockSpec((1, 1), lambda b, n: (0, 0)),
        ],
        out_shape=[
            jax.ShapeDtypeStruct((_B, _C, _HW), jnp.float32),
            jax.ShapeDtypeStruct((_B, _NB, 1, _T), jnp.int32),
            jax.ShapeDtypeStruct((1, 1), jnp.float32),
        ],
    )(x, weight)
    quant_out = out.reshape(_B, _C, _H, _W)
    loss_s = loss[0, 0] * (1.2 / (_B * _C * _HW))
    encoding_indices = idx.reshape(_B, _H, _W)
    return quant_out, loss_s, encoding_indices


# speedup vs baseline: 2.8932x; 1.1302x over previous
import jax
import jax.numpy as jnp
from jax.experimental import pallas as pl
from jax.experimental.pallas import tpu as pltpu

_B, _C, _H, _W, _K = 8, 64, 64, 64, 1024
_HW = _H * _W
_T = 4096
_NB = _HW // _T


def _vq_block(x_ref, w_ref, out_ref, idx_ref, loss_ref, ahi_ref, alo_ref,
              b2_ref):
    b = pl.program_id(0)
    nb = pl.program_id(1)

    @pl.when(jnp.logical_and(b == 0, nb == 0))
    def _build_consts():
        w = w_ref[...]
        ones = jnp.ones((_K, 1), jnp.float32)
        kcol = jax.lax.broadcasted_iota(jnp.int32, (_K, 1), 0).astype(
            jnp.float32)
        a = jnp.concatenate([w, ones, kcol], axis=1)
        a_hi = a.astype(jnp.bfloat16)
        ahi_ref[...] = a_hi
        alo_ref[...] = (a - a_hi.astype(jnp.float32)).astype(jnp.bfloat16)
        b2_ref[...] = jnp.sum(w * w, axis=1, keepdims=True)
        loss_ref[...] = jnp.zeros_like(loss_ref)

    x = x_ref[0]
    w2 = w_ref[...] + w_ref[...]
    s2 = jnp.dot(w2, x, preferred_element_type=jnp.float32)
    a2 = jnp.sum(x * x, axis=0, keepdims=True)
    m = (a2 + b2_ref[...]) - s2
    minm = jnp.min(m, axis=0, keepdims=True)
    mask = m == minm
    oh = mask.astype(jnp.bfloat16)
    dn = (((0,), (0,)), ((), ()))
    g66 = (jax.lax.dot_general(ahi_ref[...], oh, dn,
                               preferred_element_type=jnp.float32)
           + jax.lax.dot_general(alo_ref[...], oh, dn,
                                 preferred_element_type=jnp.float32))
    cnt = g66[_C:_C + 1, :]
    out_ref[0] = g66[:_C, :]
    idx_ref[0, 0] = g66[_C + 1:_C + 2, :].astype(jnp.int32)

    @pl.when(jnp.sum(cnt) > _T + 0.5)
    def _resolve_ties():
        kiota = jax.lax.broadcasted_iota(jnp.int32, m.shape, 0).astype(
            jnp.float32)
        idxf = jnp.min(jnp.where(mask, kiota, float(_K)), axis=0,
                       keepdims=True)
        oh1 = (kiota == idxf).astype(jnp.bfloat16)
        g1 = (jax.lax.dot_general(ahi_ref[...], oh1, dn,
                                  preferred_element_type=jnp.float32)
              + jax.lax.dot_general(alo_ref[...], oh1, dn,
                                    preferred_element_type=jnp.float32))
        out_ref[0] = g1[:_C, :]
        idx_ref[0, 0] = idxf.astype(jnp.int32)

    loss_ref[...] += jnp.sum(minm, axis=(0, 1), keepdims=True)


def kernel(quant_input, weight):
    x = quant_input.reshape(_B, _C, _HW)
    out, idx, loss = pl.pallas_call(
        _vq_block,
        grid=(_B, _NB),
        in_specs=[
            pl.BlockSpec((1, _C, _T), lambda b, n: (b, 0, n)),
            pl.BlockSpec((_K, _C), lambda b, n: (0, 0)),
        ],
        out_specs=[
            pl.BlockSpec((1, _C, _T), lambda b, n: (b, 0, n)),
            pl.BlockSpec((1, 1, 1, _T), lambda b, n: (b, n, 0, 0)),
            pl.BlockSpec((1, 1), lambda b, n: (0, 0)),
        ],
        out_shape=[
            jax.ShapeDtypeStruct((_B, _C, _HW), jnp.float32),
            jax.ShapeDtypeStruct((_B, _NB, 1, _T), jnp.int32),
            jax.ShapeDtypeStruct((1, 1), jnp.float32),
        ],
        scratch_shapes=[
            pltpu.VMEM((_K, _C + 2), jnp.bfloat16),
            pltpu.VMEM((_K, _C + 2), jnp.bfloat16),
            pltpu.VMEM((_K, 1), jnp.float32),
        ],
    )(x, weight)
    quant_out = out.reshape(_B, _C, _H, _W)
    loss_s = loss[0, 0] * (1.2 / (_B * _C * _HW))
    encoding_indices = idx.reshape(_B, _H, _W)
    return quant_out, loss_s, encoding_indices


# speedup vs baseline: 3.4110x; 1.1790x over previous
import jax
import jax.numpy as jnp
from jax.experimental import pallas as pl
from jax.experimental.pallas import tpu as pltpu

_B, _C, _H, _W, _K = 8, 64, 64, 64, 1024
_HW = _H * _W
_T = 4096
_NB = _HW // _T


def _vq_block(x_ref, w_ref, out_ref, idx_ref, loss_ref, a_ref, b2_ref):
    b = pl.program_id(0)
    nb = pl.program_id(1)

    @pl.when(jnp.logical_and(b == 0, nb == 0))
    def _build_consts():
        w = w_ref[...]
        ones = jnp.ones((_K, 1), jnp.float32)
        kio = jax.lax.broadcasted_iota(jnp.int32, (_K, 1), 0)
        khi = (kio // 128).astype(jnp.float32)
        klo = (kio % 128).astype(jnp.float32)
        a = jnp.concatenate([w, ones, khi, klo], axis=1)
        a_ref[...] = a.astype(jnp.bfloat16)
        b2_ref[...] = jnp.sum(w * w, axis=1, keepdims=True)
        loss_ref[...] = jnp.zeros_like(loss_ref)

    x = x_ref[0]
    w2 = w_ref[...] + w_ref[...]
    s2 = jnp.dot(w2, x, preferred_element_type=jnp.float32)
    a2 = jnp.sum(x * x, axis=0, keepdims=True)
    m = (a2 + b2_ref[...]) - s2
    minm = jnp.min(m, axis=0, keepdims=True)
    mask = m == minm
    oh = mask.astype(jnp.bfloat16)
    dn = (((0,), (0,)), ((), ()))
    g67 = jax.lax.dot_general(a_ref[...], oh, dn,
                              preferred_element_type=jnp.float32)
    cnt = g67[_C:_C + 1, :]
    out_ref[0] = g67[:_C, :]
    idx_ref[0, 0] = (g67[_C + 1:_C + 2, :] * 128.0
                     + g67[_C + 2:_C + 3, :]).astype(jnp.int32)

    @pl.when(jnp.sum(cnt) > _T + 0.5)
    def _resolve_ties():
        kiota = jax.lax.broadcasted_iota(jnp.int32, m.shape, 0).astype(
            jnp.float32)
        idxf = jnp.min(jnp.where(mask, kiota, float(_K)), axis=0,
                       keepdims=True)
        oh1 = (kiota == idxf).astype(jnp.bfloat16)
        g1 = jax.lax.dot_general(a_ref[...], oh1, dn,
                                 preferred_element_type=jnp.float32)
        out_ref[0] = g1[:_C, :]
        idx_ref[0, 0] = idxf.astype(jnp.int32)

    loss_ref[...] += jnp.sum(minm, axis=(0, 1), keepdims=True)


def kernel(quant_input, weight):
    x = quant_input.reshape(_B, _C, _HW)
    out, idx, loss = pl.pallas_call(
        _vq_block,
        grid=(_B, _NB),
        in_specs=[
            pl.BlockSpec((1, _C, _T), lambda b, n: (b, 0, n)),
            pl.BlockSpec((_K, _C), lambda b, n: (0, 0)),
        ],
        out_specs=[
            pl.BlockSpec((1, _C, _T), lambda b, n: (b, 0, n)),
            pl.BlockSpec((1, 1, 1, _T), lambda b, n: (b, n, 0, 0)),
            pl.BlockSpec((1, 1), lambda b, n: (0, 0)),
        ],
        out_shape=[
            jax.ShapeDtypeStruct((_B, _C, _HW), jnp.float32),
            jax.ShapeDtypeStruct((_B, _NB, 1, _T), jnp.int32),
            jax.ShapeDtypeStruct((1, 1), jnp.float32),
        ],
        scratch_shapes=[
            pltpu.VMEM((_K, _C + 3), jnp.bfloat16),
            pltpu.VMEM((_K, 1), jnp.float32),
        ],
    )(x, weight)
    quant_out = out.reshape(_B, _C, _H, _W)
    loss_s = loss[0, 0] * (1.2 / (_B * _C * _HW))
    encoding_indices = idx.reshape(_B, _H, _W)
    return quant_out, loss_s, encoding_indices


# speedup vs baseline: 5.1742x; 1.5169x over previous
import jax
import jax.numpy as jnp
from jax.experimental import pallas as pl
from jax.experimental.pallas import tpu as pltpu

_B, _C, _H, _W, _K = 8, 64, 64, 64, 1024
_HW = _H * _W
_T = 4096
_NB = _HW // _T


def _vq_block(x_ref, w_ref, out_ref, idx_ref, loss_ref, a_ref, b2_ref):
    b = pl.program_id(0)
    nb = pl.program_id(1)

    @pl.when(jnp.logical_and(b == 0, nb == 0))
    def _build_consts():
        w = w_ref[...]
        ones = jnp.ones((_K, 1), jnp.float32)
        kio = jax.lax.broadcasted_iota(jnp.int32, (_K, 1), 0)
        khi = (kio // 128).astype(jnp.float32)
        klo = (kio % 128).astype(jnp.float32)
        a = jnp.concatenate([w, ones, khi, klo], axis=1)
        a_ref[...] = a.astype(jnp.bfloat16)
        b2_ref[...] = jnp.sum(w * w, axis=1, keepdims=True)
        loss_ref[...] = jnp.zeros_like(loss_ref)

    x = x_ref[0].reshape(_C, _T)
    w2 = w_ref[...] + w_ref[...]
    s2 = jnp.dot(w2, x, preferred_element_type=jnp.float32)
    a2 = jnp.sum(x * x, axis=0, keepdims=True)
    m = (a2 + b2_ref[...]) - s2
    minm = jnp.min(m, axis=0, keepdims=True)
    mask = m == minm
    oh = mask.astype(jnp.bfloat16)
    dn = (((0,), (0,)), ((), ()))
    g67 = jax.lax.dot_general(a_ref[...], oh, dn,
                              preferred_element_type=jnp.float32)
    cnt = g67[_C:_C + 1, :]
    out_ref[0] = g67[:_C, :].reshape(_C, _H, _W)
    idx_ref[0] = (g67[_C + 1:_C + 2, :] * 128.0
                  + g67[_C + 2:_C + 3, :]).astype(jnp.int32).reshape(_H, _W)

    @pl.when(jnp.sum(cnt) > _T + 0.5)
    def _resolve_ties():
        kiota = jax.lax.broadcasted_iota(jnp.int32, m.shape, 0).astype(
            jnp.float32)
        idxf = jnp.min(jnp.where(mask, kiota, float(_K)), axis=0,
                       keepdims=True)
        oh1 = (kiota == idxf).astype(jnp.bfloat16)
        g1 = jax.lax.dot_general(a_ref[...], oh1, dn,
                                 preferred_element_type=jnp.float32)
        out_ref[0] = g1[:_C, :].reshape(_C, _H, _W)
        idx_ref[0] = idxf.astype(jnp.int32).reshape(_H, _W)

    loss_ref[...] += jnp.sum(minm, axis=(0, 1), keepdims=True)


def kernel(quant_input, weight):
    quant_out, idx, loss = pl.pallas_call(
        _vq_block,
        grid=(_B, _NB),
        in_specs=[
            pl.BlockSpec((1, _C, _H, _W), lambda b, n: (b, 0, n, 0)),
            pl.BlockSpec((_K, _C), lambda b, n: (0, 0)),
        ],
        out_specs=[
            pl.BlockSpec((1, _C, _H, _W), lambda b, n: (b, 0, n, 0)),
            pl.BlockSpec((1, _H, _W), lambda b, n: (b, n, 0)),
            pl.BlockSpec((1, 1), lambda b, n: (0, 0)),
        ],
        out_shape=[
            jax.ShapeDtypeStruct((_B, _C, _H, _W), jnp.float32),
            jax.ShapeDtypeStruct((_B, _H, _W), jnp.int32),
            jax.ShapeDtypeStruct((1, 1), jnp.float32),
        ],
        scratch_shapes=[
            pltpu.VMEM((_K, _C + 3), jnp.bfloat16),
            pltpu.VMEM((_K, 1), jnp.float32),
        ],
    )(quant_input, weight)
    loss_s = loss[0, 0] * (1.2 / (_B * _C * _HW))
    return quant_out, loss_s, idx
